# baseline (speedup 1.0000x reference)
import functools

import jax
import jax.numpy as jnp
from jax import lax
from jax.experimental import pallas as pl
from jax.experimental.pallas import tpu as pltpu
from jax.experimental.pallas import tpu_sc as plsc

_SIZES = (60, 60, 24, 7, 31, 12, 366)
_HIDDEN = 128
_NC, _NS = 2, 16
_NW = _NC * _NS
_ROWS_A, _ROWS_B, _ROWS_C = 3600, 24 * 7 * 31, 12 * 366
_ROWS_ALL = _ROWS_A + _ROWS_B + _ROWS_C
_DEPTH = 6


def _build_product_tables(t0, t1, t2, t3, t4, t5, t6):
    def body(t0r, t1r, t2r, t3r, t4r, t5r, t6r, out):
        a = t0r[:][:, None, :] + t1r[:][None, :, :]
        b = (t2r[:][:, None, None, :] + t3r[:][None, :, None, :]
             ) + t4r[:][None, None, :, :]
        c = t5r[:][:, None, :] + t6r[:][None, :, :]
        out[pl.ds(0, _ROWS_A)] = a.reshape(_ROWS_A, _HIDDEN)
        out[pl.ds(_ROWS_A, _ROWS_B)] = b.reshape(_ROWS_B, _HIDDEN)
        out[pl.ds(_ROWS_A + _ROWS_B, _ROWS_C)] = c.reshape(_ROWS_C, _HIDDEN)

    return pl.pallas_call(
        body,
        out_shape=jax.ShapeDtypeStruct((_ROWS_ALL, _HIDDEN), jnp.float32),
    )(t0, t1, t2, t3, t4, t5, t6)


@functools.lru_cache(maxsize=None)
def _make_transpose(n_total: int):
    nblk = 2048

    def body(ts_ref, o_ref):
        t = ts_ref[...].T
        o_ref[pl.ds(0, 7)] = t
        o_ref[pl.ds(7, 1)] = jnp.zeros((1, t.shape[1]), jnp.float32)

    return pl.pallas_call(
        body,
        grid=(n_total // nblk,),
        in_specs=[pl.BlockSpec((nblk, 7), lambda i: (i, 0))],
        out_specs=pl.BlockSpec((8, nblk), lambda i: (0, i)),
        out_shape=jax.ShapeDtypeStruct((8, n_total), jnp.float32),
    )


@functools.lru_cache(maxsize=None)
def _make_sc_kernel(n_total: int):
    n_per_w = n_total // _NW
    rows = 128
    n_chunks = n_per_w // rows
    mesh = plsc.VectorSubcoreMesh(
        core_axis_name="c", subcore_axis_name="s",
        num_cores=_NC, num_subcores=_NS,
    )

    ring = 4

    @functools.partial(
        pl.kernel,
        out_type=jax.ShapeDtypeStruct((n_total, _HIDDEN), jnp.float32),
        mesh=mesh,
        scratch_types=(
            [pltpu.VMEM((8, rows), jnp.float32)] * ring
            + [pltpu.VMEM((n_chunks, 3, 128), jnp.int32)]
            + [pltpu.VMEM((rows, _HIDDEN), jnp.float32)] * _DEPTH
            + [pltpu.SemaphoreType.DMA] * ring
            + [pltpu.SemaphoreType.DMA] * _DEPTH
            + [pltpu.SemaphoreType.DMA] * _DEPTH
            + [pltpu.SemaphoreType.DMA] * _DEPTH
        ),
    )
    def sc_kernel(ts_hbm, tall, out_hbm, *rest):
        tsb = rest[:ring]
        idx_v = rest[ring]
        accs = rest[ring + 1:ring + 1 + _DEPTH]
        tsem = rest[ring + 1 + _DEPTH:2 * ring + 1 + _DEPTH]
        g0sem = rest[2 * ring + 1 + _DEPTH:2 * ring + 1 + 2 * _DEPTH]
        asem = rest[2 * ring + 1 + 2 * _DEPTH:2 * ring + 1 + 3 * _DEPTH]
        wsem = rest[2 * ring + 1 + 3 * _DEPTH:2 * ring + 1 + 4 * _DEPTH]
        wid = lax.axis_index("s") * _NC + lax.axis_index("c")
        base = wid * n_per_w

        def ts_load(c):
            p = c % ring
            return pltpu.async_copy(
                ts_hbm.at[:, pl.ds(base + c * rows, rows)], tsb[p], tsem[p])

        def compute_idx(c):
            src = tsb[c % ring]

            def idx_grp(k, carry):
                def tix(i):
                    v = src[i, pl.ds(k * 16, 16)]
                    return (v * jnp.float32(_SIZES[i])).astype(jnp.int32)

                ia = tix(0) * 60 + tix(1)
                ib = (tix(2) * 217 + tix(3) * 31 + tix(4)) + _ROWS_A
                ic = (tix(5) * 366 + tix(6)) + (_ROWS_A + _ROWS_B)
                sl = pl.ds(k * 16, 16)
                idx_v[c, 0, sl] = ia
                idx_v[c, 1, sl] = ib
                idx_v[c, 2, sl] = ic
                return carry
            lax.fori_loop(0, rows // 16, idx_grp, 0)

        def issue_g0(c):
            p = c % _DEPTH
            return pltpu.async_copy(
                tall.at[idx_v.at[c, 0]], accs[p], g0sem[p])

        def issue_adds(c):
            p = c % _DEPTH
            return [pltpu.async_copy(
                tall.at[idx_v.at[c, i]], accs[p], asem[p], add=True)
                for i in (1, 2)]

        def issue_write(c):
            p = c % _DEPTH
            return pltpu.async_copy(
                accs[p], out_hbm.at[pl.ds(base + c * rows, rows)], wsem[p])

        tdesc = [None] * ring
        for j in range(min(ring, n_chunks)):
            tdesc[j] = ts_load(j)
        for j in range(min(2, n_chunks)):
            tdesc[j].wait()
            compute_idx(j)

        gdesc = [None] * _DEPTH
        adesc = [None] * _DEPTH
        wdesc = [None] * _DEPTH
        for c in range(n_chunks + 4):
            if c < n_chunks:
                p = c % _DEPTH
                if wdesc[p] is not None:
                    wdesc[p].wait()
                    wdesc[p] = None
                gdesc[p] = issue_g0(c)
            if 2 <= c < n_chunks + 2:
                q = (c - 2) % _DEPTH
                gdesc[q].wait()
                adesc[q] = issue_adds(c - 2)
            if c >= 4:
                r = (c - 4) % _DEPTH
                for d in adesc[r]:
                    d.wait()
                wdesc[r] = issue_write(c - 4)
            if c + 2 < n_chunks:
                tdesc[(c + 2) % ring].wait()
                compute_idx(c + 2)
            if c + ring < n_chunks:
                tdesc[(c + ring) % ring] = ts_load(c + ring)
        for d in wdesc:
            if d is not None:
                d.wait()

    return sc_kernel


def kernel(timestamps, table_0, table_1, table_2, table_3, table_4,
           table_5, table_6):
    b, s, f = timestamps.shape
    n_total = b * s
    tall = _build_product_tables(
        table_0, table_1, table_2, table_3, table_4, table_5, table_6)
    ts_t = _make_transpose(n_total)(timestamps.reshape(n_total, f))
    out = _make_sc_kernel(n_total)(ts_t, tall)
    return out.reshape(b, s, _HIDDEN)


# speedup vs baseline: 1.4233x; 1.4233x over previous
import functools

import jax
import jax.numpy as jnp
from jax import lax
from jax.experimental import pallas as pl
from jax.experimental.pallas import tpu as pltpu
from jax.experimental.pallas import tpu_sc as plsc

_SIZES = (60, 60, 24, 7, 31, 12, 366)
_HIDDEN = 128
_NC, _NS = 2, 16
_NW = _NC * _NS
_ROWS_A, _ROWS_B, _ROWS_C = 3600, 24 * 7 * 31, 12 * 366
_ROWS_ALL = _ROWS_A + _ROWS_B + _ROWS_C
_DEPTH = 6


def _build_product_tables(t0, t1, t2, t3, t4, t5, t6):
    def body(t0r, t1r, t2r, t3r, t4r, t5r, t6r, out):
        a = t0r[:][:, None, :] + t1r[:][None, :, :]
        b = (t2r[:][:, None, None, :] + t3r[:][None, :, None, :]
             ) + t4r[:][None, None, :, :]
        c = t5r[:][:, None, :] + t6r[:][None, :, :]
        out[pl.ds(0, _ROWS_A)] = a.reshape(_ROWS_A, _HIDDEN)
        out[pl.ds(_ROWS_A, _ROWS_B)] = b.reshape(_ROWS_B, _HIDDEN)
        out[pl.ds(_ROWS_A + _ROWS_B, _ROWS_C)] = c.reshape(_ROWS_C, _HIDDEN)

    return pl.pallas_call(
        body,
        out_shape=jax.ShapeDtypeStruct((_ROWS_ALL, _HIDDEN), jnp.float32),
    )(t0, t1, t2, t3, t4, t5, t6)


@functools.lru_cache(maxsize=None)
def _make_sc_kernel(n_total: int):
    n_per_w = n_total // _NW
    rows = 128
    n_chunks = n_per_w // rows
    mesh = plsc.VectorSubcoreMesh(
        core_axis_name="c", subcore_axis_name="s",
        num_cores=_NC, num_subcores=_NS,
    )

    ring = 4

    @functools.partial(
        pl.kernel,
        out_type=jax.ShapeDtypeStruct((n_total, _HIDDEN), jnp.float32),
        mesh=mesh,
        scratch_types=(
            [pltpu.VMEM((7, rows), jnp.float32)] * ring
            + [pltpu.VMEM((n_chunks, 3, 128), jnp.int32)]
            + [pltpu.VMEM((rows, _HIDDEN), jnp.float32)] * _DEPTH
            + [pltpu.SemaphoreType.DMA] * ring
            + [pltpu.SemaphoreType.DMA] * _DEPTH
            + [pltpu.SemaphoreType.DMA] * _DEPTH
            + [pltpu.SemaphoreType.DMA] * _DEPTH
        ),
    )
    def sc_kernel(ts_hbm, tall, out_hbm, *rest):
        tsb = rest[:ring]
        idx_v = rest[ring]
        accs = rest[ring + 1:ring + 1 + _DEPTH]
        tsem = rest[ring + 1 + _DEPTH:2 * ring + 1 + _DEPTH]
        g0sem = rest[2 * ring + 1 + _DEPTH:2 * ring + 1 + 2 * _DEPTH]
        asem = rest[2 * ring + 1 + 2 * _DEPTH:2 * ring + 1 + 3 * _DEPTH]
        wsem = rest[2 * ring + 1 + 3 * _DEPTH:2 * ring + 1 + 4 * _DEPTH]
        wid = lax.axis_index("s") * _NC + lax.axis_index("c")
        base = wid * n_per_w

        def ts_load(c):
            p = c % ring
            return pltpu.async_copy(
                ts_hbm.at[:, pl.ds(base + c * rows, rows)], tsb[p], tsem[p])

        def compute_idx(c):
            src = tsb[c % ring]

            def idx_grp(k, carry):
                def tix(i):
                    v = src[i, pl.ds(k * 16, 16)]
                    return (v * jnp.float32(_SIZES[i])).astype(jnp.int32)

                ia = tix(0) * 60 + tix(1)
                ib = (tix(2) * 217 + tix(3) * 31 + tix(4)) + _ROWS_A
                ic = (tix(5) * 366 + tix(6)) + (_ROWS_A + _ROWS_B)
                sl = pl.ds(k * 16, 16)
                idx_v[c, 0, sl] = ia
                idx_v[c, 1, sl] = ib
                idx_v[c, 2, sl] = ic
                return carry
            lax.fori_loop(0, rows // 16, idx_grp, 0)

        def issue_g0(c):
            p = c % _DEPTH
            return pltpu.async_copy(
                tall.at[idx_v.at[c, 0]], accs[p], g0sem[p])

        def issue_adds(c):
            p = c % _DEPTH
            return [pltpu.async_copy(
                tall.at[idx_v.at[c, i]], accs[p], asem[p], add=True)
                for i in (1, 2)]

        def issue_write(c):
            p = c % _DEPTH
            return pltpu.async_copy(
                accs[p], out_hbm.at[pl.ds(base + c * rows, rows)], wsem[p])

        tdesc = [None] * ring
        for j in range(min(ring, n_chunks)):
            tdesc[j] = ts_load(j)
        for j in range(min(2, n_chunks)):
            tdesc[j].wait()
            compute_idx(j)

        gdesc = [None] * _DEPTH
        adesc = [None] * _DEPTH
        wdesc = [None] * _DEPTH
        for c in range(n_chunks + 4):
            if c < n_chunks:
                p = c % _DEPTH
                if wdesc[p] is not None:
                    wdesc[p].wait()
                    wdesc[p] = None
                gdesc[p] = issue_g0(c)
            if 2 <= c < n_chunks + 2:
                q = (c - 2) % _DEPTH
                gdesc[q].wait()
                adesc[q] = issue_adds(c - 2)
            if c >= 4:
                r = (c - 4) % _DEPTH
                for d in adesc[r]:
                    d.wait()
                wdesc[r] = issue_write(c - 4)
            if c + 2 < n_chunks:
                tdesc[(c + 2) % ring].wait()
                compute_idx(c + 2)
            if c + ring < n_chunks:
                tdesc[(c + ring) % ring] = ts_load(c + ring)
        for d in wdesc:
            if d is not None:
                d.wait()

    return sc_kernel


def kernel(timestamps, table_0, table_1, table_2, table_3, table_4,
           table_5, table_6):
    b, s, f = timestamps.shape
    n_total = b * s
    tall = _build_product_tables(
        table_0, table_1, table_2, table_3, table_4, table_5, table_6)
    ts_t = timestamps.reshape(n_total, f).T
    out = _make_sc_kernel(n_total)(ts_t, tall)
    return out.reshape(b, s, _HIDDEN)
